# baseline (speedup 1.0000x reference)
import functools

import jax
import jax.numpy as jnp
from jax import lax
from jax.experimental import pallas as pl
from jax.experimental.pallas import tpu as pltpu
from jax.experimental.pallas import tpu_sc as plsc

_R = 64
_D = 128

_NC = 2
_NS = 16
_NW = _NC * _NS

_CH = 128
_NBUF = 4
_LOOK = 3


def _matmul_body(ut_ref, b_ref, w_ref):
    w_ref[...] = jax.lax.dot_general(
        ut_ref[...], b_ref[...],
        dimension_numbers=(((0,), (0,)), ((), ())),
        preferred_element_type=jnp.float32,
    )


def _compute_w(U, B):
    m = U.shape[0]
    blk = 32768
    grid = (m + blk - 1) // blk
    return pl.pallas_call(
        _matmul_body,
        grid=(grid,),
        in_specs=[
            pl.BlockSpec((_R, blk), lambda i: (0, i)),
            pl.BlockSpec((_R, _D), lambda i: (0, 0)),
        ],
        out_specs=pl.BlockSpec((blk, _D), lambda i: (i, 0)),
        out_shape=jax.ShapeDtypeStruct((m, _D), jnp.float32),
    )(U.T, B)


def _make_gather(ntot, nch):
    mesh = plsc.VectorSubcoreMesh(core_axis_name="c", subcore_axis_name="s")
    per_w = nch * _CH

    @functools.partial(
        pl.kernel,
        out_type=jax.ShapeDtypeStruct((ntot, _D), jnp.float32),
        mesh=mesh,
        compiler_params=pltpu.CompilerParams(use_tc_tiling_on_sc=False),
        scratch_types=[
            pltpu.VMEM((per_w,), jnp.int32),
            [pltpu.VMEM((_CH, _D), jnp.float32)] * _NBUF,
            [pltpu.SemaphoreType.DMA] * _NBUF,
            [pltpu.SemaphoreType.DMA] * _NBUF,
        ],
    )
    def gather(table_hbm, idx_hbm, out_hbm, idx_v, rows, gsems, osems):
        wid = lax.axis_index("s") * _NC + lax.axis_index("c")
        base = wid * per_w
        pltpu.sync_copy(idx_hbm.at[pl.ds(base, per_w)], idx_v)

        def gather_chunk(c, buf):
            pltpu.make_async_copy(
                table_hbm.at[idx_v.at[pl.ds(c * _CH, _CH)]],
                rows[buf],
                gsems[buf],
            ).start()

        def out_copy(c, buf):
            return pltpu.make_async_copy(
                rows[buf],
                out_hbm.at[pl.ds(base + c * _CH, _CH)],
                osems[buf],
            )

        for c in range(_LOOK):
            gather_chunk(c, c % _NBUF)

        def body(g, _):
            for b in range(_NBUF):
                c = 4 * g + b
                nxt = c + _LOOK
                nbuf = (b + _LOOK) % _NBUF

                @pl.when(nxt < nch)
                def _():
                    @pl.when(c >= 1)
                    def _():
                        out_copy(c - 1, nbuf).wait()

                    gather_chunk(nxt, nbuf)

                pltpu.make_async_copy(
                    table_hbm.at[idx_v.at[pl.ds(c * _CH, _CH)]],
                    rows[b],
                    gsems[b],
                ).wait()
                out_copy(c, b).start()
            return 0

        lax.fori_loop(0, nch // _NBUF, body, 0, unroll=False)
        for k in range(nch - _NBUF, nch):
            out_copy(k, k % _NBUF).wait()

    return gather


def kernel(local_ids, U, B):
    bsz, seq = local_ids.shape
    ntot = bsz * seq
    nch = ntot // (_NW * _CH)

    W = _compute_w(U, B)
    ids = local_ids.astype(jnp.int32).reshape(ntot)
    out = _make_gather(ntot, nch)(W, ids)
    return out.reshape(bsz, seq, _D)
